# baseline (speedup 1.0000x reference)
import jax
import jax.numpy as jnp
import numpy as np
from jax.experimental import pallas as pl
from jax.experimental.pallas import tpu as pltpu

_B, _N, _D, _H = 8, 65536, 3, 64
_BLK = 32768
_NB = _N // _BLK
_NR = _N * _D // 384

_SEL = np.zeros((384, 384), np.float32)
_k = np.arange(384)
_SEL[_k, 128 * (_k % 3) + _k // 3] = 1.0


def _deint_kernel(s_ref, t_ref, sel_ref, so_ref, to_ref):
    sel = sel_ref[...]
    for ref, oref in ((s_ref, so_ref), (t_ref, to_ref)):
        y = jax.lax.dot_general(ref[0], sel, (((1,), (0,)), ((), ())),
                                preferred_element_type=jnp.float32)
        for d in range(_D):
            oref[0, d] = y[:, d * 128:(d + 1) * 128]


def _mlp_kernel(s_ref, t_ref, w1c_ref, w2c_ref, b2_ref, out_ref, dacc):
    b = pl.program_id(0)
    nb = pl.program_id(1)

    @pl.when(jnp.logical_and(b == 0, nb == 0))
    def _init():
        dacc[...] = jnp.zeros_like(dacc)

    ones = jnp.ones((1, _BLK), dtype=jnp.float32)
    rhs = jnp.concatenate([s_ref[0], t_ref[0], ones], axis=0)
    h = jax.lax.dot_general(w1c_ref[...], rhs, (((1,), (0,)), ((), ())),
                            preferred_element_type=jnp.float32)
    h = jnp.maximum(h, 0.0)
    o = jax.lax.dot_general(w2c_ref[...], h, (((1,), (0,)), ((), ())),
                            preferred_element_type=jnp.float32)
    o = jnp.maximum(o + b2_ref[...], 0.0)
    diff = o[0:1, :] - o[1:2, :]
    dacc[pl.ds(b, 1), :] += jnp.sum(diff, axis=1, keepdims=True)

    @pl.when(jnp.logical_and(b == _B - 1, nb == _NB - 1))
    def _fin():
        d = dacc[...]
        out_ref[...] = jnp.mean(d * d, keepdims=True)


def kernel(student_diagrams, teacher_diagrams, W1, b1, W2, b2):
    sI = student_diagrams.reshape(_B, _NR, 384)
    tI = teacher_diagrams.reshape(_B, _NR, 384)

    sT, tT = pl.pallas_call(
        _deint_kernel,
        grid=(_B,),
        in_specs=[
            pl.BlockSpec((1, _NR, 384), lambda b: (b, 0, 0)),
            pl.BlockSpec((1, _NR, 384), lambda b: (b, 0, 0)),
            pl.BlockSpec((384, 384), lambda b: (0, 0)),
        ],
        out_specs=[
            pl.BlockSpec((1, _D, _NR, 128), lambda b: (b, 0, 0, 0)),
            pl.BlockSpec((1, _D, _NR, 128), lambda b: (b, 0, 0, 0)),
        ],
        out_shape=[
            jax.ShapeDtypeStruct((_B, _D, _NR, 128), jnp.float32),
            jax.ShapeDtypeStruct((_B, _D, _NR, 128), jnp.float32),
        ],
    )(sI, tI, jnp.asarray(_SEL))
    sF = sT.reshape(_B, _D, _N)
    tF = tT.reshape(_B, _D, _N)

    z3 = jnp.zeros((_H, _D), jnp.float32)
    top = jnp.concatenate([W1.T, z3, b1[:, None]], axis=1)
    bot = jnp.concatenate([z3, W1.T, b1[:, None]], axis=1)
    w1c = jnp.concatenate([top, bot], axis=0)
    z64 = jnp.zeros((1, _H), jnp.float32)
    w2c = jnp.concatenate([
        jnp.concatenate([W2.T, z64], axis=1),
        jnp.concatenate([z64, W2.T], axis=1),
    ], axis=0)
    b2r = b2.reshape(1, 1)

    out = pl.pallas_call(
        _mlp_kernel,
        grid=(_B, _NB),
        in_specs=[
            pl.BlockSpec((1, _D, _BLK), lambda b, nb: (b, 0, nb)),
            pl.BlockSpec((1, _D, _BLK), lambda b, nb: (b, 0, nb)),
            pl.BlockSpec((_H * 2, _D * 2 + 1), lambda b, nb: (0, 0)),
            pl.BlockSpec((2, _H * 2), lambda b, nb: (0, 0)),
            pl.BlockSpec((1, 1), lambda b, nb: (0, 0)),
        ],
        out_specs=pl.BlockSpec((1, 1), lambda b, nb: (0, 0)),
        out_shape=jax.ShapeDtypeStruct((1, 1), jnp.float32),
        scratch_shapes=[
            pltpu.VMEM((_B, 1), jnp.float32),
        ],
    )(sF, tF, w1c, w2c, b2r)
    return out[0, 0]


# speedup vs baseline: 8.8574x; 8.8574x over previous
import jax
import jax.numpy as jnp
from jax.experimental import pallas as pl
from jax.experimental.pallas import tpu as pltpu

_B, _N, _D, _H = 8, 65536, 3, 64
_BLK = 8192
_NB = _N // _BLK


def _mlp_kernel(s_ref, t_ref, w1c_ref, w2c_ref, b2_ref, out_ref, dacc):
    nb = pl.program_id(0)

    @pl.when(nb == 0)
    def _init():
        dacc[...] = jnp.zeros_like(dacc)

    ones = jnp.ones((1, _BLK), dtype=jnp.float32)
    w1c = w1c_ref[...]
    w2c = w2c_ref[...]
    b2v = b2_ref[...]
    for b in range(_B):
        rhs = jnp.concatenate([s_ref[:, b, :], t_ref[:, b, :], ones], axis=0)
        h = jax.lax.dot_general(w1c, rhs, (((1,), (0,)), ((), ())),
                                preferred_element_type=jnp.float32)
        h = jnp.maximum(h, 0.0)
        o = jax.lax.dot_general(w2c, h, (((1,), (0,)), ((), ())),
                                preferred_element_type=jnp.float32)
        o = jnp.maximum(o + b2v, 0.0)
        diff = o[0:1, :] - o[1:2, :]
        dacc[b:b + 1, :] += jnp.sum(diff, axis=1, keepdims=True)

    @pl.when(nb == _NB - 1)
    def _fin():
        d = dacc[...]
        out_ref[...] = jnp.mean(d * d, keepdims=True)


def kernel(student_diagrams, teacher_diagrams, W1, b1, W2, b2):
    sP = jnp.transpose(student_diagrams, (2, 0, 1))
    tP = jnp.transpose(teacher_diagrams, (2, 0, 1))

    z3 = jnp.zeros((_H, _D), jnp.float32)
    top = jnp.concatenate([W1.T, z3, b1[:, None]], axis=1)
    bot = jnp.concatenate([z3, W1.T, b1[:, None]], axis=1)
    w1c = jnp.concatenate([top, bot], axis=0)
    z64 = jnp.zeros((1, _H), jnp.float32)
    w2c = jnp.concatenate([
        jnp.concatenate([W2.T, z64], axis=1),
        jnp.concatenate([z64, W2.T], axis=1),
    ], axis=0)
    b2r = b2.reshape(1, 1)

    out = pl.pallas_call(
        _mlp_kernel,
        grid=(_NB,),
        in_specs=[
            pl.BlockSpec((_D, _B, _BLK), lambda nb: (0, 0, nb)),
            pl.BlockSpec((_D, _B, _BLK), lambda nb: (0, 0, nb)),
            pl.BlockSpec((_H * 2, _D * 2 + 1), lambda nb: (0, 0)),
            pl.BlockSpec((2, _H * 2), lambda nb: (0, 0)),
            pl.BlockSpec((1, 1), lambda nb: (0, 0)),
        ],
        out_specs=pl.BlockSpec((1, 1), lambda nb: (0, 0)),
        out_shape=jax.ShapeDtypeStruct((1, 1), jnp.float32),
        scratch_shapes=[
            pltpu.VMEM((_B, 1), jnp.float32),
        ],
    )(sP, tP, w1c, w2c, b2r)
    return out[0, 0]


# speedup vs baseline: 8.9975x; 1.0158x over previous
import jax
import jax.numpy as jnp
from jax.experimental import pallas as pl
from jax.experimental.pallas import tpu as pltpu

_B, _N, _D, _H = 8, 65536, 3, 64
_BLK = 16384
_NB = _N // _BLK


def _mlp_kernel(s_ref, t_ref, w1c_ref, w2c_ref, b2_ref, out_ref, dacc):
    nb = pl.program_id(0)

    @pl.when(nb == 0)
    def _init():
        dacc[...] = jnp.zeros_like(dacc)

    ones = jnp.ones((1, _BLK), dtype=jnp.float32)
    w1c = w1c_ref[...]
    w2c = w2c_ref[...]
    b2v = b2_ref[...]
    for b in range(_B):
        rhs = jnp.concatenate([s_ref[:, b, :], t_ref[:, b, :], ones], axis=0)
        h = jax.lax.dot_general(w1c, rhs, (((1,), (0,)), ((), ())),
                                preferred_element_type=jnp.float32)
        h = jnp.maximum(h, 0.0)
        o = jax.lax.dot_general(w2c, h, (((1,), (0,)), ((), ())),
                                preferred_element_type=jnp.float32)
        o = jnp.maximum(o + b2v, 0.0)
        diff = o[0:1, :] - o[1:2, :]
        dacc[b:b + 1, :] += jnp.sum(diff, axis=1, keepdims=True)

    @pl.when(nb == _NB - 1)
    def _fin():
        d = dacc[...]
        out_ref[...] = jnp.mean(d * d, keepdims=True)


def kernel(student_diagrams, teacher_diagrams, W1, b1, W2, b2):
    sP = jnp.transpose(student_diagrams, (2, 0, 1))
    tP = jnp.transpose(teacher_diagrams, (2, 0, 1))

    z3 = jnp.zeros((_H, _D), jnp.float32)
    top = jnp.concatenate([W1.T, z3, b1[:, None]], axis=1)
    bot = jnp.concatenate([z3, W1.T, b1[:, None]], axis=1)
    w1c = jnp.concatenate([top, bot], axis=0)
    z64 = jnp.zeros((1, _H), jnp.float32)
    w2c = jnp.concatenate([
        jnp.concatenate([W2.T, z64], axis=1),
        jnp.concatenate([z64, W2.T], axis=1),
    ], axis=0)
    b2r = b2.reshape(1, 1)

    out = pl.pallas_call(
        _mlp_kernel,
        grid=(_NB,),
        in_specs=[
            pl.BlockSpec((_D, _B, _BLK), lambda nb: (0, 0, nb)),
            pl.BlockSpec((_D, _B, _BLK), lambda nb: (0, 0, nb)),
            pl.BlockSpec((_H * 2, _D * 2 + 1), lambda nb: (0, 0)),
            pl.BlockSpec((2, _H * 2), lambda nb: (0, 0)),
            pl.BlockSpec((1, 1), lambda nb: (0, 0)),
        ],
        out_specs=pl.BlockSpec((1, 1), lambda nb: (0, 0)),
        out_shape=jax.ShapeDtypeStruct((1, 1), jnp.float32),
        scratch_shapes=[
            pltpu.VMEM((_B, 1), jnp.float32),
        ],
    )(sP, tP, w1c, w2c, b2r)
    return out[0, 0]
